# baseline (speedup 1.0000x reference)
import functools

import jax
import jax.numpy as jnp
from jax import lax
from jax.experimental import pallas as pl
from jax.experimental.pallas import tpu as pltpu
from jax.experimental.pallas import tpu_sc as plsc

F = 256
FC = 64
NCHUNK = 4
NLAYERS = 3
NB = 64
NCLS = 10
N0, N1, N2 = 10000, 20000, 5000
E0, E1, E2 = 160000, 60000, 20000
K = 128
NTILES = 16


def _rup(x, m):
    return (x + m - 1) // m * m


E0P, E1P, E2P = (_rup(E0, 2 * NTILES * K), _rup(E1, 2 * NTILES * K),
                 _rup(E2, 2 * NTILES * K))
N0P, N1P, N2P = _rup(N0, NTILES * K), _rup(N1, NTILES * K), _rup(N2, NTILES * K)
ACC_ROWS = N1 + 64


def _sc_body(x0c, x1c, x2c, e0, e1u, e1d, e2, bat0, bat1, bat2, zer,
             pooled, a0, a1, a2, b0, b1, b2,
             acc, bvec, eb0, eb1, rs0, ra0, rs1, ra1,
             sem_e0, sem_e1, sem_g0, sem_g1, sem_w0, sem_w1):
    cid = lax.axis_index("c")
    tid = lax.axis_index("s")
    slots = ((eb0, rs0, ra0, sem_e0, sem_g0, sem_w0),
             (eb1, rs1, ra1, sem_e1, sem_g1, sem_w1))

    def copy_rows(src2d, dst2d, n):
        per = (n // NTILES) // 8 * 8
        rem = n - per * NTILES
        if per:
            pltpu.sync_copy(src2d.at[pl.ds(tid * per, per)],
                            dst2d.at[pl.ds(tid * per, per)])
        if rem:
            @pl.when(tid == 0)
            def _():
                pltpu.sync_copy(src2d.at[pl.ds(per * NTILES, rem)],
                                dst2d.at[pl.ds(per * NTILES, rem)])

    def edge_pass(e_hbm, ep, xs, xa, c):
        m = ep // K // NTILES
        base = tid * m

        def g1_issue(p):
            eb, rs, ra, _, sg, _2 = slots[p]
            pltpu.async_copy(xs.at[c].at[eb.at[0]], rs, sg)

        def g1_wait(p):
            eb, rs, ra, _, sg, _2 = slots[p]
            pltpu.make_async_copy(xs.at[c].at[eb.at[0]], rs, sg).wait()

        def g2_issue(p):
            eb, rs, ra, _, sg, _2 = slots[p]
            pltpu.async_copy(xa.at[c].at[eb.at[1]], rs, sg, add=True)

        def g2_wait(p):
            eb, rs, ra, _, sg, _2 = slots[p]
            pltpu.make_async_copy(xa.at[c].at[eb.at[1]], rs, sg).wait()

        def e_issue(b, p):
            eb, _, _, se, _2, _3 = slots[p]
            pltpu.async_copy(e_hbm.at[b], eb, se)

        def e_wait(b, p):
            eb, _, _, se, _2, _3 = slots[p]
            pltpu.make_async_copy(e_hbm.at[b], eb, se).wait()

        def s_issue(p):
            eb, rs, ra, _, _2, sw = slots[p]
            pltpu.async_copy(rs, acc.at[eb.at[2]], sw, add=True)

        def s_wait(p):
            eb, rs, ra, _, _2, sw = slots[p]
            pltpu.make_async_copy(rs, acc.at[eb.at[2]], sw).wait()

        pltpu.sync_copy(e_hbm.at[base], eb0)
        g1_issue(0)
        e_issue(base + 1, 1)

        def pair(ii, carry):
            bb = base + 2 * ii
            g1_wait(0)
            g2_issue(0)
            e_wait(bb + 1, 1)
            g1_issue(1)
            g2_wait(0)
            s_issue(0)
            g1_wait(1)
            g2_issue(1)
            s_wait(0)
            e_issue(bb + 2, 0)
            g2_wait(1)
            s_issue(1)
            e_wait(bb + 2, 0)
            g1_issue(0)
            s_wait(1)
            e_issue(bb + 3, 1)
            return carry

        lax.fori_loop(0, m // 2 - 1, pair, 0)
        g1_wait(0)
        g2_issue(0)
        e_wait(base + m - 1, 1)
        g1_issue(1)
        g2_wait(0)
        s_issue(0)
        g1_wait(1)
        g2_issue(1)
        s_wait(0)
        g2_wait(1)
        s_issue(1)
        s_wait(1)

    def phase(xd, out, c, n_dst, passes):
        copy_rows(xd.at[c], acc, n_dst)
        plsc.subcore_barrier()
        for (e_hbm, ep, xs, xa) in passes:
            edge_pass(e_hbm, ep, xs, xa, c)
        plsc.subcore_barrier()
        copy_rows(acc, out.at[c], n_dst)
        plsc.subcore_barrier()

    ins = (x0c, x1c, x2c)
    pong = (a0, a1, a2)
    ping = (b0, b1, b2)
    seq = [(ins, pong), (pong, ping), (ping, pong)]
    for (xi, xo) in seq:
        for j in range(2):
            c = cid * 2 + j
            phase(xi[0], xo[0], c, N0, [(e0, E0P, xi[0], xi[1])])
            phase(xi[1], xo[1], c, N1, [(e1u, E1P, xi[1], xi[2]),
                                        (e1d, E1P, xi[1], xi[0])])
            phase(xi[2], xo[2], c, N2, [(e2, E2P, xi[2], xi[1])])

    xf = pong
    for j in range(2):
        c = cid * 2 + j
        copy_rows(zer, acc, 80)
        plsc.subcore_barrier()
        for (xb, bt, npad) in ((xf[0], bat0, N0P), (xf[1], bat1, N1P),
                               (xf[2], bat2, N2P)):
            m = npad // K // NTILES

            def pblk(i, carry, xb=xb, bt=bt, m=m, c=c):
                base = (tid * m + i) * K
                pltpu.sync_copy(bt.at[pl.ds(base, K)], bvec)
                pltpu.sync_copy(xb.at[c].at[pl.ds(base, K)], rs0)
                pltpu.sync_copy(rs0, acc.at[bvec], add=True)
                return carry

            lax.fori_loop(0, m, pblk, 0)
        plsc.subcore_barrier()
        copy_rows(acc, pooled.at[c], NB)
        plsc.subcore_barrier()


_sc_kernel = functools.partial(
    pl.kernel,
    out_type=[
        jax.ShapeDtypeStruct((NCHUNK, NB, FC), jnp.float32),
        jax.ShapeDtypeStruct((NCHUNK, N0P, FC), jnp.float32),
        jax.ShapeDtypeStruct((NCHUNK, N1P, FC), jnp.float32),
        jax.ShapeDtypeStruct((NCHUNK, N2P, FC), jnp.float32),
        jax.ShapeDtypeStruct((NCHUNK, N0P, FC), jnp.float32),
        jax.ShapeDtypeStruct((NCHUNK, N1P, FC), jnp.float32),
        jax.ShapeDtypeStruct((NCHUNK, N2P, FC), jnp.float32),
    ],
    mesh=plsc.VectorSubcoreMesh(core_axis_name="c", subcore_axis_name="s"),
    compiler_params=pltpu.CompilerParams(use_tc_tiling_on_sc=False),
    scratch_types=[
        pltpu.VMEM_SHARED((ACC_ROWS, FC), jnp.float32),
        pltpu.VMEM((K,), jnp.int32),
        pltpu.VMEM((3, K), jnp.int32),
        pltpu.VMEM((3, K), jnp.int32),
        pltpu.VMEM((K, FC), jnp.float32),
        pltpu.VMEM((K, FC), jnp.float32),
        pltpu.VMEM((K, FC), jnp.float32),
        pltpu.VMEM((K, FC), jnp.float32),
        pltpu.SemaphoreType.DMA,
        pltpu.SemaphoreType.DMA,
        pltpu.SemaphoreType.DMA,
        pltpu.SemaphoreType.DMA,
        pltpu.SemaphoreType.DMA,
        pltpu.SemaphoreType.DMA,
    ],
)(_sc_body)


def _mm_body(p_ref, w_ref, b_ref, o_ref):
    o_ref[...] = (jnp.dot(p_ref[...], w_ref[...],
                          preferred_element_type=jnp.float32) + b_ref[...])


_tc_matmul = pl.pallas_call(
    _mm_body,
    out_shape=jax.ShapeDtypeStruct((NB, NCLS), jnp.float32),
)


def _edges(src, attr, dst, ep, ndst):
    pad = ep - src.shape[0]
    z = jnp.zeros((pad,), jnp.int32)
    src = jnp.concatenate([src, z])
    attr = jnp.concatenate([attr, z])
    dst = jnp.concatenate([dst, jnp.full((pad,), ndst, jnp.int32)])
    return jnp.stack([src, attr, dst]).reshape(3, ep // K, K).transpose(1, 0, 2)


def _chunked(x):
    return x.reshape(x.shape[0], NCHUNK, FC).transpose(1, 0, 2)


def _padbat(bt, npad):
    return jnp.concatenate([bt, jnp.full((npad - bt.shape[0],), NB, jnp.int32)])


def kernel(x0, x1, x2, up_index0, shared_cob0, up_index1, shared_cob1,
           down_index1, shared_face1, down_index2, shared_face2,
           batch0, batch1, batch2, W, b):
    x0c, x1c, x2c = _chunked(x0), _chunked(x1), _chunked(x2)
    e0 = _edges(up_index0[0], shared_cob0, up_index0[1], E0P, N0)
    e1u = _edges(up_index1[0], shared_cob1, up_index1[1], E1P, N1)
    e1d = _edges(down_index1[0], shared_face1, down_index1[1], E1P, N1)
    e2 = _edges(down_index2[0], shared_face2, down_index2[1], E2P, N2)
    bat0, bat1, bat2 = (_padbat(batch0, N0P), _padbat(batch1, N1P),
                        _padbat(batch2, N2P))
    zer = jnp.zeros((80, FC), jnp.float32)
    outs = _sc_kernel(x0c, x1c, x2c, e0, e1u, e1d, e2, bat0, bat1, bat2, zer)
    pooled = outs[0].transpose(1, 0, 2).reshape(NB, F)
    return _tc_matmul(pooled, W.T, b.reshape(1, NCLS))


# speedup vs baseline: 1.3421x; 1.3421x over previous
import functools

import jax
import jax.numpy as jnp
from jax import lax
from jax.experimental import pallas as pl
from jax.experimental.pallas import tpu as pltpu
from jax.experimental.pallas import tpu_sc as plsc

F = 256
FC = 64
NCHUNK = 4
NLAYERS = 3
NB = 64
NCLS = 10
N0, N1, N2 = 10000, 20000, 5000
E0, E1, E2 = 160000, 60000, 20000
K = 128
NTILES = 16


def _rup(x, m):
    return (x + m - 1) // m * m


E0P, E1P, E2P = (_rup(E0, 2 * NTILES * K), _rup(E1, 2 * NTILES * K),
                 _rup(E2, 2 * NTILES * K))
N0P, N1P, N2P = _rup(N0, NTILES * K), _rup(N1, NTILES * K), _rup(N2, NTILES * K)
ACC_ROWS = N1 + 64


def _sc_body(x0c, x1c, x2c, e0, e1u, e1d, e2, bat0, bat1, bat2, zer,
             pooled, a0, a1, a2, b0, b1, b2,
             acc, bvec, eb0, eb1, db0, db1, rs0, ra0, rs1, ra1,
             sem_e0, sem_e1, sem_g0, sem_g1, sem_h0, sem_h1, sem_w0, sem_w1):
    cid = lax.axis_index("c")
    tid = lax.axis_index("s")
    slots = ((eb0, db0, rs0, ra0, sem_e0, sem_g0, sem_h0, sem_w0),
             (eb1, db1, rs1, ra1, sem_e1, sem_g1, sem_h1, sem_w1))

    def copy_rows(src2d, dst2d, n):
        per = (n // NTILES) // 8 * 8
        rem = n - per * NTILES
        if per:
            pltpu.sync_copy(src2d.at[pl.ds(tid * per, per)],
                            dst2d.at[pl.ds(tid * per, per)])
        if rem:
            @pl.when(tid == 0)
            def _():
                pltpu.sync_copy(src2d.at[pl.ds(per * NTILES, rem)],
                                dst2d.at[pl.ds(per * NTILES, rem)])

    def edge_pass(e_hbm, ep, xs, xa, c):
        m = ep // K // NTILES
        base = tid * m

        def g_issue(p):
            eb, db, rs, ra, se, sg, sh, sw = slots[p]
            pltpu.async_copy(xs.at[c].at[eb.at[0]], rs, sg)
            pltpu.async_copy(xa.at[c].at[eb.at[1]], ra, sh)

        def g_wait_db(p):
            eb, db, rs, ra, se, sg, sh, sw = slots[p]
            pltpu.make_async_copy(xs.at[c].at[eb.at[0]], rs, sg).wait()
            pltpu.make_async_copy(xa.at[c].at[eb.at[1]], ra, sh).wait()
            for q in range(K // 16):
                sl = pl.ds(q * 16, 16)
                db[sl] = eb[2, sl]

        def e_issue(b, p):
            eb, db, _, _2, se, _3, _4, _5 = slots[p]
            pltpu.async_copy(e_hbm.at[b], eb, se)

        def e_wait(b, p):
            eb, db, _, _2, se, _3, _4, _5 = slots[p]
            pltpu.make_async_copy(e_hbm.at[b], eb, se).wait()

        def s_issue(p):
            eb, db, rs, ra, _, _2, _3, sw = slots[p]
            pltpu.async_copy(rs, acc.at[db], sw, add=True)
            pltpu.async_copy(ra, acc.at[db], sw, add=True)

        def s_wait(p):
            eb, db, rs, ra, _, _2, _3, sw = slots[p]
            pltpu.make_async_copy(rs, acc.at[db], sw).wait()
            pltpu.make_async_copy(ra, acc.at[db], sw).wait()

        pltpu.sync_copy(e_hbm.at[base], eb0)
        g_issue(0)
        e_issue(base + 1, 1)

        def pair(ii, carry):
            bb = base + 2 * ii
            g_wait_db(0)
            e_issue(bb + 2, 0)
            s_issue(0)
            e_wait(bb + 1, 1)
            g_issue(1)
            s_wait(0)
            e_wait(bb + 2, 0)
            g_issue(0)
            g_wait_db(1)
            e_issue(bb + 3, 1)
            s_issue(1)
            s_wait(1)
            return carry

        lax.fori_loop(0, m // 2 - 1, pair, 0)
        g_wait_db(0)
        s_issue(0)
        e_wait(base + m - 1, 1)
        g_issue(1)
        s_wait(0)
        g_wait_db(1)
        s_issue(1)
        s_wait(1)

    def phase(xd, out, c, n_dst, passes):
        copy_rows(xd.at[c], acc, n_dst)
        plsc.subcore_barrier()
        for (e_hbm, ep, xs, xa) in passes:
            edge_pass(e_hbm, ep, xs, xa, c)
        plsc.subcore_barrier()
        copy_rows(acc, out.at[c], n_dst)
        plsc.subcore_barrier()

    ins = (x0c, x1c, x2c)
    pong = (a0, a1, a2)
    ping = (b0, b1, b2)
    seq = [(ins, pong), (pong, ping), (ping, pong)]
    for (xi, xo) in seq:
        for j in range(2):
            c = cid * 2 + j
            phase(xi[0], xo[0], c, N0, [(e0, E0P, xi[0], xi[1])])
            phase(xi[1], xo[1], c, N1, [(e1u, E1P, xi[1], xi[2]),
                                        (e1d, E1P, xi[1], xi[0])])
            phase(xi[2], xo[2], c, N2, [(e2, E2P, xi[2], xi[1])])

    xf = pong
    for j in range(2):
        c = cid * 2 + j
        copy_rows(zer, acc, 80)
        plsc.subcore_barrier()
        for (xb, bt, npad) in ((xf[0], bat0, N0P), (xf[1], bat1, N1P),
                               (xf[2], bat2, N2P)):
            m = npad // K // NTILES

            def pblk(i, carry, xb=xb, bt=bt, m=m, c=c):
                base = (tid * m + i) * K
                pltpu.sync_copy(bt.at[pl.ds(base, K)], bvec)
                pltpu.sync_copy(xb.at[c].at[pl.ds(base, K)], rs0)
                pltpu.sync_copy(rs0, acc.at[bvec], add=True)
                return carry

            lax.fori_loop(0, m, pblk, 0)
        plsc.subcore_barrier()
        copy_rows(acc, pooled.at[c], NB)
        plsc.subcore_barrier()


_sc_kernel = functools.partial(
    pl.kernel,
    out_type=[
        jax.ShapeDtypeStruct((NCHUNK, NB, FC), jnp.float32),
        jax.ShapeDtypeStruct((NCHUNK, N0P, FC), jnp.float32),
        jax.ShapeDtypeStruct((NCHUNK, N1P, FC), jnp.float32),
        jax.ShapeDtypeStruct((NCHUNK, N2P, FC), jnp.float32),
        jax.ShapeDtypeStruct((NCHUNK, N0P, FC), jnp.float32),
        jax.ShapeDtypeStruct((NCHUNK, N1P, FC), jnp.float32),
        jax.ShapeDtypeStruct((NCHUNK, N2P, FC), jnp.float32),
    ],
    mesh=plsc.VectorSubcoreMesh(core_axis_name="c", subcore_axis_name="s"),
    compiler_params=pltpu.CompilerParams(use_tc_tiling_on_sc=False),
    scratch_types=[
        pltpu.VMEM_SHARED((ACC_ROWS, FC), jnp.float32),
        pltpu.VMEM((K,), jnp.int32),
        pltpu.VMEM((3, K), jnp.int32),
        pltpu.VMEM((3, K), jnp.int32),
        pltpu.VMEM((K,), jnp.int32),
        pltpu.VMEM((K,), jnp.int32),
        pltpu.VMEM((K, FC), jnp.float32),
        pltpu.VMEM((K, FC), jnp.float32),
        pltpu.VMEM((K, FC), jnp.float32),
        pltpu.VMEM((K, FC), jnp.float32),
        pltpu.SemaphoreType.DMA,
        pltpu.SemaphoreType.DMA,
        pltpu.SemaphoreType.DMA,
        pltpu.SemaphoreType.DMA,
        pltpu.SemaphoreType.DMA,
        pltpu.SemaphoreType.DMA,
        pltpu.SemaphoreType.DMA,
        pltpu.SemaphoreType.DMA,
    ],
)(_sc_body)


def _mm_body(p_ref, w_ref, b_ref, o_ref):
    o_ref[...] = (jnp.dot(p_ref[...], w_ref[...],
                          preferred_element_type=jnp.float32) + b_ref[...])


_tc_matmul = pl.pallas_call(
    _mm_body,
    out_shape=jax.ShapeDtypeStruct((NB, NCLS), jnp.float32),
)


def _edges(src, attr, dst, ep, ndst):
    pad = ep - src.shape[0]
    z = jnp.zeros((pad,), jnp.int32)
    src = jnp.concatenate([src, z])
    attr = jnp.concatenate([attr, z])
    dst = jnp.concatenate([dst, jnp.full((pad,), ndst, jnp.int32)])
    return jnp.stack([src, attr, dst]).reshape(3, ep // K, K).transpose(1, 0, 2)


def _chunked(x):
    return x.reshape(x.shape[0], NCHUNK, FC).transpose(1, 0, 2)


def _padbat(bt, npad):
    return jnp.concatenate([bt, jnp.full((npad - bt.shape[0],), NB, jnp.int32)])


def kernel(x0, x1, x2, up_index0, shared_cob0, up_index1, shared_cob1,
           down_index1, shared_face1, down_index2, shared_face2,
           batch0, batch1, batch2, W, b):
    x0c, x1c, x2c = _chunked(x0), _chunked(x1), _chunked(x2)
    e0 = _edges(up_index0[0], shared_cob0, up_index0[1], E0P, N0)
    e1u = _edges(up_index1[0], shared_cob1, up_index1[1], E1P, N1)
    e1d = _edges(down_index1[0], shared_face1, down_index1[1], E1P, N1)
    e2 = _edges(down_index2[0], shared_face2, down_index2[1], E2P, N2)
    bat0, bat1, bat2 = (_padbat(batch0, N0P), _padbat(batch1, N1P),
                        _padbat(batch2, N2P))
    zer = jnp.zeros((80, FC), jnp.float32)
    outs = _sc_kernel(x0c, x1c, x2c, e0, e1u, e1d, e2, bat0, bat1, bat2, zer)
    pooled = outs[0].transpose(1, 0, 2).reshape(NB, F)
    return _tc_matmul(pooled, W.T, b.reshape(1, NCLS))
